# baseline (speedup 1.0000x reference)
import functools

import jax
import jax.numpy as jnp
from jax import lax
from jax.experimental import pallas as pl
from jax.experimental.pallas import tpu as pltpu
from jax.experimental.pallas import tpu_sc as plsc

B = 64
S = 2048
H = 64
NC = 2
NS = 16
NW = NC * NS
SBLK = 128
NSB = S // SBLK
BPW = B // (NW // NSB)
LANES = 16
G = SBLK // LANES


def _emb_body(x_hbm, wt_hbm, posT_hbm, out_hbm,
              xbuf, pbufT, gbuf, obuf, gsem):
  wid = lax.axis_index("s") * NC + lax.axis_index("c")
  sb = wid % NSB
  s0 = sb * SBLK
  b0 = (wid // NSB) * BPW

  pltpu.sync_copy(posT_hbm.at[:, pl.ds(s0, SBLK)], pbufT)

  rows = [lax.iota(jnp.int32, LANES) + g * LANES for g in range(G)]

  @pl.loop(0, BPW)
  def _chunk(i):
    b = b0 + i
    pltpu.sync_copy(x_hbm.at[b, pl.ds(s0, SBLK)], xbuf)

    for g in range(G):
      vg = xbuf[pl.ds(g * LANES, LANES)]
      for k in range(LANES):
        pltpu.async_copy(wt_hbm.at[vg[k]], gbuf.at[g * LANES + k], gsem)
    pltpu.make_async_copy(wt_hbm.at[pl.ds(0, SBLK)], gbuf, gsem).wait()

    @pl.loop(0, H, unroll=8)
    def _h(h):
      hvec = jnp.full((LANES,), 0, jnp.int32) + h
      for g in range(G):
        sl = pl.ds(g * LANES, LANES)
        vec = plsc.load_gather(gbuf, [rows[g], hvec])
        obuf[h, sl] = vec + pbufT[h, sl]

    pltpu.sync_copy(obuf, out_hbm.at[b, :, pl.ds(s0, SBLK)])


@jax.jit
def _emb(x, word_table, posT):
  mesh = plsc.VectorSubcoreMesh(
      core_axis_name="c", subcore_axis_name="s", num_cores=NC, num_subcores=NS
  )
  return pl.kernel(
      _emb_body,
      out_type=jax.ShapeDtypeStruct((B, H, S), jnp.float32),
      mesh=mesh,
      scratch_types=[
          pltpu.VMEM((SBLK,), jnp.int32),
          pltpu.VMEM((H, SBLK), jnp.float32),
          pltpu.VMEM((SBLK, H), jnp.float32),
          pltpu.VMEM((H, SBLK), jnp.float32),
          pltpu.SemaphoreType.DMA,
      ],
      compiler_params=pltpu.CompilerParams(needs_layout_passes=False),
  )(x, word_table, posT)


def kernel(x, word_table, pos_table):
  x = x.astype(jnp.int32)
  posT = jnp.swapaxes(pos_table, 0, 1)
  out = _emb(x, word_table, posT)
  return jnp.swapaxes(out, 1, 2)


# speedup vs baseline: 1.0022x; 1.0022x over previous
import functools

import jax
import jax.numpy as jnp
from jax import lax
from jax.experimental import pallas as pl
from jax.experimental.pallas import tpu as pltpu
from jax.experimental.pallas import tpu_sc as plsc

B = 64
S = 2048
H = 64
NC = 2
NS = 16
NW = NC * NS
SBLK = 128
NSB = S // SBLK
BPW = B // (NW // NSB)
LANES = 16
G = SBLK // LANES


def _emb_body(x_hbm, wt_hbm, posT_hbm, out_hbm,
              xbuf, pbufT, gbuf, obuf, gsem):
  wid = lax.axis_index("s") * NC + lax.axis_index("c")
  sb = wid % NSB
  s0 = sb * SBLK
  b0 = (wid // NSB) * BPW

  pltpu.sync_copy(posT_hbm.at[:, pl.ds(s0, SBLK)], pbufT)

  rows = [lax.iota(jnp.int32, LANES) + g * LANES for g in range(G)]
  zero16 = lax.iota(jnp.int32, LANES) * 0

  @pl.loop(0, BPW)
  def _chunk(i):
    b = b0 + i
    pltpu.sync_copy(x_hbm.at[b, pl.ds(s0, SBLK)], xbuf)

    for g in range(G):
      vg = xbuf[pl.ds(g * LANES, LANES)]
      for k in range(LANES):
        pltpu.async_copy(wt_hbm.at[vg[k]], gbuf.at[g * LANES + k], gsem)
    pltpu.make_async_copy(wt_hbm.at[pl.ds(0, SBLK)], gbuf, gsem).wait()

    for h in range(H):
      hvec = zero16 + h
      for g in range(G):
        sl = pl.ds(g * LANES, LANES)
        vec = plsc.load_gather(gbuf, [rows[g], hvec])
        obuf[h, sl] = vec + pbufT[h, sl]

    pltpu.sync_copy(obuf, out_hbm.at[b, :, pl.ds(s0, SBLK)])


@jax.jit
def _emb(x, word_table, posT):
  mesh = plsc.VectorSubcoreMesh(
      core_axis_name="c", subcore_axis_name="s", num_cores=NC, num_subcores=NS
  )
  return pl.kernel(
      _emb_body,
      out_type=jax.ShapeDtypeStruct((B, H, S), jnp.float32),
      mesh=mesh,
      scratch_types=[
          pltpu.VMEM((SBLK,), jnp.int32),
          pltpu.VMEM((H, SBLK), jnp.float32),
          pltpu.VMEM((SBLK, H), jnp.float32),
          pltpu.VMEM((H, SBLK), jnp.float32),
          pltpu.SemaphoreType.DMA,
      ],
      compiler_params=pltpu.CompilerParams(needs_layout_passes=False),
  )(x, word_table, posT)


def kernel(x, word_table, pos_table):
  x = x.astype(jnp.int32)
  posT = jnp.swapaxes(pos_table, 0, 1)
  out = _emb(x, word_table, posT)
  return jnp.swapaxes(out, 1, 2)


# speedup vs baseline: 1.0622x; 1.0599x over previous
import functools

import jax
import jax.numpy as jnp
from jax import lax
from jax.experimental import pallas as pl
from jax.experimental.pallas import tpu as pltpu
from jax.experimental.pallas import tpu_sc as plsc

B = 64
S = 2048
H = 64
NC = 2
NS = 16
NW = NC * NS
SBLK = 128
NSB = S // SBLK
BPW = B // (NW // NSB)
LANES = 16
G = SBLK // LANES
OBYTES = H * SBLK * 4


def _emb_body(x_hbm, wt_hbm, posT_hbm, out_hbm,
              xbuf, pbufT, gbuf0, gbuf1, obuf0, obuf1,
              gsem0, gsem1, osem0, osem1):
  wid = lax.axis_index("s") * NC + lax.axis_index("c")
  sb = wid % NSB
  s0 = sb * SBLK
  b0 = (wid // NSB) * BPW

  pltpu.sync_copy(posT_hbm.at[:, pl.ds(s0, SBLK)], pbufT)

  rows = [lax.iota(jnp.int32, LANES) + g * LANES for g in range(G)]
  zero16 = lax.iota(jnp.int32, LANES) * 0

  def load_and_issue(b, gb, gsem):
    pltpu.sync_copy(x_hbm.at[b, pl.ds(s0, SBLK)], xbuf)
    @pl.loop(0, SBLK, step=LANES)
    def _grp(r0):
      vg = xbuf[pl.ds(r0, LANES)]
      for k in range(LANES):
        pltpu.async_copy(wt_hbm.at[vg[k]], gb.at[r0 + k], gsem)

  def wait_gather(gb, gsem):
    pltpu.make_async_copy(wt_hbm.at[pl.ds(0, SBLK)], gb, gsem).wait()

  def wait_out(ob, osem):
    pltpu.make_async_copy(ob, out_hbm.at[b0, :, pl.ds(s0, SBLK)],
                          osem).wait()

  def extract(gb, ob):
    @pl.loop(0, H, unroll=8)
    def _h(h):
      hvec = zero16 + h
      for g in range(G):
        sl = pl.ds(g * LANES, LANES)
        vec = plsc.load_gather(gb, [rows[g], hvec])
        ob[h, sl] = vec + pbufT[h, sl]

  def start_out(b, ob, osem):
    pltpu.async_copy(ob, out_hbm.at[b, :, pl.ds(s0, SBLK)], osem)

  bufs = ((gbuf0, gsem0, obuf0, osem0), (gbuf1, gsem1, obuf1, osem1))

  load_and_issue(b0, gbuf0, gsem0)
  wait_gather(gbuf0, gsem0)
  load_and_issue(b0 + 1, gbuf1, gsem1)
  extract(gbuf0, obuf0)
  start_out(b0, obuf0, osem0)
  wait_gather(gbuf1, gsem1)
  load_and_issue(b0 + 2, gbuf0, gsem0)
  extract(gbuf1, obuf1)
  start_out(b0 + 1, obuf1, osem1)

  @pl.loop(2, BPW - 2, step=2)
  def _pair(i):
    for k in range(2):
      gb, gsem, ob, osem = bufs[k]
      ngb, ngsem, _, _ = bufs[1 - k]
      b = b0 + i + k
      wait_gather(gb, gsem)
      load_and_issue(b + 1, ngb, ngsem)
      wait_out(ob, osem)
      extract(gb, ob)
      start_out(b, ob, osem)

  wait_gather(gbuf0, gsem0)
  load_and_issue(b0 + BPW - 1, gbuf1, gsem1)
  wait_out(obuf0, osem0)
  extract(gbuf0, obuf0)
  start_out(b0 + BPW - 2, obuf0, osem0)

  wait_gather(gbuf1, gsem1)
  wait_out(obuf1, osem1)
  extract(gbuf1, obuf1)
  start_out(b0 + BPW - 1, obuf1, osem1)

  wait_out(obuf0, osem0)
  wait_out(obuf1, osem1)


@jax.jit
def _emb(x, word_table, posT):
  mesh = plsc.VectorSubcoreMesh(
      core_axis_name="c", subcore_axis_name="s", num_cores=NC, num_subcores=NS
  )
  return pl.kernel(
      _emb_body,
      out_type=jax.ShapeDtypeStruct((B, H, S), jnp.float32),
      mesh=mesh,
      scratch_types=[
          pltpu.VMEM((SBLK,), jnp.int32),
          pltpu.VMEM((H, SBLK), jnp.float32),
          pltpu.VMEM((SBLK, H), jnp.float32),
          pltpu.VMEM((SBLK, H), jnp.float32),
          pltpu.VMEM((H, SBLK), jnp.float32),
          pltpu.VMEM((H, SBLK), jnp.float32),
          pltpu.SemaphoreType.DMA,
          pltpu.SemaphoreType.DMA,
          pltpu.SemaphoreType.DMA,
          pltpu.SemaphoreType.DMA,
      ],
      compiler_params=pltpu.CompilerParams(needs_layout_passes=False),
  )(x, word_table, posT)


def kernel(x, word_table, pos_table):
  x = x.astype(jnp.int32)
  posT = jnp.swapaxes(pos_table, 0, 1)
  out = _emb(x, word_table, posT)
  return jnp.swapaxes(out, 1, 2)


# speedup vs baseline: 1.0724x; 1.0096x over previous
import functools

import jax
import jax.numpy as jnp
from jax import lax
from jax.experimental import pallas as pl
from jax.experimental.pallas import tpu as pltpu
from jax.experimental.pallas import tpu_sc as plsc

B = 64
S = 2048
H = 64
NC = 2
NS = 16
NW = NC * NS
SBLK = 128
NSB = S // SBLK
BPW = B // (NW // NSB)
LANES = 16
G = SBLK // LANES
OBYTES = H * SBLK * 4


def _emb_body(x_hbm, wt_hbm, posT_hbm, out_hbm,
              xbuf, pbufT, gbuf0, gbuf1, obuf0, obuf1,
              gsem0, gsem1, osem0, osem1):
  wid = lax.axis_index("s") * NC + lax.axis_index("c")
  sb = wid % NSB
  s0 = sb * SBLK
  b0 = (wid // NSB) * BPW

  pltpu.sync_copy(posT_hbm.at[:, pl.ds(s0, SBLK)], pbufT)

  rows = [lax.iota(jnp.int32, LANES) + g * LANES for g in range(G)]
  zero16 = lax.iota(jnp.int32, LANES) * 0

  def load_and_issue(b, gb, gsem):
    pltpu.sync_copy(x_hbm.at[b, pl.ds(s0, SBLK)], xbuf)
    @pl.loop(0, SBLK, step=LANES)
    def _grp(r0):
      vg = xbuf[pl.ds(r0, LANES)]
      for k in range(LANES):
        pltpu.async_copy(wt_hbm.at[vg[k]], gb.at[r0 + k], gsem)

  def wait_gather(gb, gsem):
    pltpu.make_async_copy(wt_hbm.at[pl.ds(0, SBLK)], gb, gsem).wait()

  def wait_out(ob, osem):
    pltpu.make_async_copy(ob, out_hbm.at[b0, :, pl.ds(s0, SBLK)],
                          osem).wait()

  def extract(gb, ob):
    @pl.loop(0, H, unroll=8)
    def _h(h):
      hvec = zero16 + h
      for g in range(G):
        sl = pl.ds(g * LANES, LANES)
        vec = plsc.load_gather(gb, [rows[g], hvec])
        ob[h, sl] = vec + pbufT[h, sl]

  def start_out(b, ob, osem):
    pltpu.async_copy(ob, out_hbm.at[b, :, pl.ds(s0, SBLK)], osem)

  def fused_issue_extract(b_next, gb, ob, ngb, ngsem):
    pltpu.sync_copy(x_hbm.at[b_next, pl.ds(s0, SBLK)], xbuf)
    @pl.loop(0, G)
    def _j(j):
      r0 = j * LANES
      vg = xbuf[pl.ds(r0, LANES)]
      for k in range(LANES):
        pltpu.async_copy(wt_hbm.at[vg[k]], ngb.at[r0 + k], ngsem)
      h0 = j * (H // G)
      for hh in range(H // G):
        h = h0 + hh
        hvec = zero16 + h
        for g in range(G):
          sl = pl.ds(g * LANES, LANES)
          vec = plsc.load_gather(gb, [rows[g], hvec])
          ob[h, sl] = vec + pbufT[h, sl]

  bufs = ((gbuf0, gsem0, obuf0, osem0), (gbuf1, gsem1, obuf1, osem1))

  load_and_issue(b0, gbuf0, gsem0)
  wait_gather(gbuf0, gsem0)
  load_and_issue(b0 + 1, gbuf1, gsem1)
  extract(gbuf0, obuf0)
  start_out(b0, obuf0, osem0)
  wait_gather(gbuf1, gsem1)
  load_and_issue(b0 + 2, gbuf0, gsem0)
  extract(gbuf1, obuf1)
  start_out(b0 + 1, obuf1, osem1)

  @pl.loop(2, BPW - 2, step=2)
  def _pair(i):
    for k in range(2):
      gb, gsem, ob, osem = bufs[k]
      ngb, ngsem, _, _ = bufs[1 - k]
      b = b0 + i + k
      wait_gather(gb, gsem)
      wait_out(ob, osem)
      fused_issue_extract(b + 1, gb, ob, ngb, ngsem)
      start_out(b, ob, osem)

  wait_gather(gbuf0, gsem0)
  load_and_issue(b0 + BPW - 1, gbuf1, gsem1)
  wait_out(obuf0, osem0)
  extract(gbuf0, obuf0)
  start_out(b0 + BPW - 2, obuf0, osem0)

  wait_gather(gbuf1, gsem1)
  wait_out(obuf1, osem1)
  extract(gbuf1, obuf1)
  start_out(b0 + BPW - 1, obuf1, osem1)

  wait_out(obuf0, osem0)
  wait_out(obuf1, osem1)


@jax.jit
def _emb(x, word_table, posT):
  mesh = plsc.VectorSubcoreMesh(
      core_axis_name="c", subcore_axis_name="s", num_cores=NC, num_subcores=NS
  )
  return pl.kernel(
      _emb_body,
      out_type=jax.ShapeDtypeStruct((B, H, S), jnp.float32),
      mesh=mesh,
      scratch_types=[
          pltpu.VMEM((SBLK,), jnp.int32),
          pltpu.VMEM((H, SBLK), jnp.float32),
          pltpu.VMEM((SBLK, H), jnp.float32),
          pltpu.VMEM((SBLK, H), jnp.float32),
          pltpu.VMEM((H, SBLK), jnp.float32),
          pltpu.VMEM((H, SBLK), jnp.float32),
          pltpu.SemaphoreType.DMA,
          pltpu.SemaphoreType.DMA,
          pltpu.SemaphoreType.DMA,
          pltpu.SemaphoreType.DMA,
      ],
      compiler_params=pltpu.CompilerParams(needs_layout_passes=False),
  )(x, word_table, posT)


def kernel(x, word_table, pos_table):
  x = x.astype(jnp.int32)
  posT = jnp.swapaxes(pos_table, 0, 1)
  out = _emb(x, word_table, posT)
  return jnp.swapaxes(out, 1, 2)
